# baseline (speedup 1.0000x reference)
import functools

import jax
import jax.numpy as jnp
from jax import lax
from jax.experimental import pallas as pl
from jax.experimental.pallas import tpu as pltpu
from jax.experimental.pallas import tpu_sc as plsc

_NC = 2
_NS = 16
_NW = _NC * _NS
_LANES = 16
_N0 = 6
_N1 = 1
_K = 4


@functools.partial(jax.jit, static_argnums=(2, 3, 4))
def _gather_mean(idx_r, features, B, chunk_rows, S):
    N, D = features.shape
    C = chunk_rows
    BP = (idx_r.shape[0] // S)
    inv_s = jnp.float32(1.0 / S)
    tail = B % C

    mesh = plsc.VectorSubcoreMesh(
        core_axis_name="c", subcore_axis_name="s",
        num_cores=_NC, num_subcores=_NS,
    )

    @functools.partial(
        pl.kernel,
        mesh=mesh,
        out_type=jax.ShapeDtypeStruct((B, D), jnp.float32),
        scratch_types=[
            pltpu.VMEM((S * C,), jnp.int32),
            pltpu.VMEM((S * C,), jnp.int32),
            pltpu.VMEM((C, D), jnp.float32),
            pltpu.VMEM((C, D), jnp.float32),
            pltpu.SemaphoreType.DMA,
            pltpu.SemaphoreType.DMA,
            pltpu.SemaphoreType.DMA,
            pltpu.SemaphoreType.DMA,
        ],
    )
    def body(idx_hbm, feat_hbm, out_hbm,
             idx_a, idx_b, acc_a, acc_b, s0a, s0b, sma, smb):
        c = lax.axis_index("c")
        s = lax.axis_index("s")
        n_my = jnp.where(
            c == 0,
            _N0 + jnp.where(s < _K, 1, 0),
            jnp.where(s < _NS - _K, _N1, 0),
        )
        chunk0 = jnp.where(
            c == 0,
            s * _N0 + jnp.minimum(s, _K),
            _NS * _N0 + _K + s * _N1,
        )

        def prep(i_ref, a_ref, s0, gid):
            pltpu.sync_copy(idx_hbm.at[pl.ds(gid * (S * C), S * C)], i_ref)
            pltpu.async_copy(feat_hbm.at[i_ref.at[pl.ds(0, C)]], a_ref, s0)

        def launch_adds(i_ref, a_ref, s0, sm):
            pltpu.make_async_copy(
                feat_hbm.at[i_ref.at[pl.ds(0, C)]], a_ref, s0
            ).wait()
            for j in range(1, S):
                pltpu.async_copy(
                    feat_hbm.at[i_ref.at[pl.ds(j * C, C)]], a_ref, sm,
                    add=True,
                )

        def finish(i_ref, a_ref, sm, gid):
            for j in range(1, S):
                pltpu.make_async_copy(
                    feat_hbm.at[i_ref.at[pl.ds(j * C, C)]], a_ref, sm
                ).wait()

            def scale_row(r, carry2):
                for k in range(D // _LANES):
                    col = pl.ds(k * _LANES, _LANES)
                    a_ref[r, col] = a_ref[r, col] * inv_s
                return carry2

            lax.fori_loop(0, C, scale_row, 0)
            row0 = gid * C

            @pl.when(row0 + C <= B)
            def _full():
                pltpu.sync_copy(a_ref, out_hbm.at[pl.ds(row0, C)])

            if tail:
                @pl.when(row0 == B - tail)
                def _partial():
                    pltpu.sync_copy(
                        a_ref.at[pl.ds(0, tail)],
                        out_hbm.at[pl.ds(B - tail, tail)],
                    )

        @pl.when(n_my > 0)
        def _prologue():
            prep(idx_a, acc_a, s0a, chunk0)

        def pair(i, carry):
            g0 = 2 * i
            launch_adds(idx_a, acc_a, s0a, sma)

            @pl.when(i > 0)
            def _finish_prev():
                finish(idx_b, acc_b, smb, chunk0 + g0 - 1)

            prep(idx_b, acc_b, s0b, chunk0 + g0 + 1)
            launch_adds(idx_b, acc_b, s0b, smb)
            finish(idx_a, acc_a, sma, chunk0 + g0)

            @pl.when(g0 + 2 < n_my)
            def _prep_next():
                prep(idx_a, acc_a, s0a, chunk0 + g0 + 2)

            return carry

        lax.fori_loop(0, n_my // 2, pair, 0)

        @pl.when(n_my % 2 == 1)
        def _epilogue_odd():
            launch_adds(idx_a, acc_a, s0a, sma)

            @pl.when(n_my > 1)
            def _finish_b():
                finish(idx_b, acc_b, smb, chunk0 + n_my - 2)

            finish(idx_a, acc_a, sma, chunk0 + n_my - 1)

        @pl.when((n_my % 2 == 0) & (n_my > 0))
        def _epilogue_even():
            finish(idx_b, acc_b, smb, chunk0 + n_my - 1)

    return body(idx_r, features)


def kernel(nodes, neigh_idx, num_sample, features):
    B, S = neigh_idx.shape
    del nodes, num_sample
    C = 448
    per = _NS * (_N0 + _N1) * C
    BP = ((B + per - 1) // per) * per
    idx = neigh_idx.astype(jnp.int32)
    if BP != B:
        idx = jnp.concatenate([idx, jnp.zeros((BP - B, S), jnp.int32)])
    idx_r = jnp.reshape(
        jnp.transpose(jnp.reshape(idx, (BP // C, C, S)), (0, 2, 1)), (-1,)
    )
    return _gather_mean(idx_r, features, B, C, S)


# speedup vs baseline: 1.0079x; 1.0079x over previous
import functools

import jax
import jax.numpy as jnp
from jax import lax
from jax.experimental import pallas as pl
from jax.experimental.pallas import tpu as pltpu
from jax.experimental.pallas import tpu_sc as plsc

_NC = 2
_NS = 16
_NW = _NC * _NS
_LANES = 16
_N0 = 6
_N1 = 1
_K = 0


@functools.partial(jax.jit, static_argnums=(2, 3, 4))
def _gather_mean(idx_r, features, B, chunk_rows, S):
    N, D = features.shape
    C = chunk_rows
    BP = (idx_r.shape[0] // S)
    inv_s = jnp.float32(1.0 / S)
    tail = B % C

    mesh = plsc.VectorSubcoreMesh(
        core_axis_name="c", subcore_axis_name="s",
        num_cores=_NC, num_subcores=_NS,
    )

    @functools.partial(
        pl.kernel,
        mesh=mesh,
        out_type=jax.ShapeDtypeStruct((B, D), jnp.float32),
        scratch_types=[
            pltpu.VMEM((S * C,), jnp.int32),
            pltpu.VMEM((S * C,), jnp.int32),
            pltpu.VMEM((C, D), jnp.float32),
            pltpu.VMEM((C, D), jnp.float32),
            pltpu.SemaphoreType.DMA,
            pltpu.SemaphoreType.DMA,
            pltpu.SemaphoreType.DMA,
            pltpu.SemaphoreType.DMA,
        ],
    )
    def body(idx_hbm, feat_hbm, out_hbm,
             idx_a, idx_b, acc_a, acc_b, s0a, s0b, sma, smb):
        c = lax.axis_index("c")
        s = lax.axis_index("s")
        n_my = jnp.where(
            c == 0,
            _N0 + jnp.where(s < _K, 1, 0),
            jnp.where(s < _NS - _K, _N1, 0),
        )
        chunk0 = jnp.where(
            c == 0,
            s * _N0 + jnp.minimum(s, _K),
            _NS * _N0 + _K + s * _N1,
        )

        def prep(i_ref, a_ref, s0, gid):
            pltpu.sync_copy(idx_hbm.at[pl.ds(gid * (S * C), S * C)], i_ref)
            pltpu.async_copy(feat_hbm.at[i_ref.at[pl.ds(0, C)]], a_ref, s0)

        def launch_adds(i_ref, a_ref, s0, sm):
            pltpu.make_async_copy(
                feat_hbm.at[i_ref.at[pl.ds(0, C)]], a_ref, s0
            ).wait()
            for j in range(1, S):
                pltpu.async_copy(
                    feat_hbm.at[i_ref.at[pl.ds(j * C, C)]], a_ref, sm,
                    add=True,
                )

        def finish(i_ref, a_ref, sm, gid):
            for j in range(1, S):
                pltpu.make_async_copy(
                    feat_hbm.at[i_ref.at[pl.ds(j * C, C)]], a_ref, sm
                ).wait()

            def scale_row(r, carry2):
                for k in range(D // _LANES):
                    col = pl.ds(k * _LANES, _LANES)
                    a_ref[r, col] = a_ref[r, col] * inv_s
                return carry2

            lax.fori_loop(0, C, scale_row, 0)
            row0 = gid * C

            @pl.when(row0 + C <= B)
            def _full():
                pltpu.sync_copy(a_ref, out_hbm.at[pl.ds(row0, C)])

            if tail:
                @pl.when(row0 == B - tail)
                def _partial():
                    pltpu.sync_copy(
                        a_ref.at[pl.ds(0, tail)],
                        out_hbm.at[pl.ds(B - tail, tail)],
                    )

        @pl.when(n_my > 0)
        def _prologue():
            prep(idx_a, acc_a, s0a, chunk0)

        def pair(i, carry):
            g0 = 2 * i
            launch_adds(idx_a, acc_a, s0a, sma)

            @pl.when(i > 0)
            def _finish_prev():
                finish(idx_b, acc_b, smb, chunk0 + g0 - 1)

            prep(idx_b, acc_b, s0b, chunk0 + g0 + 1)
            launch_adds(idx_b, acc_b, s0b, smb)
            finish(idx_a, acc_a, sma, chunk0 + g0)

            @pl.when(g0 + 2 < n_my)
            def _prep_next():
                prep(idx_a, acc_a, s0a, chunk0 + g0 + 2)

            return carry

        lax.fori_loop(0, n_my // 2, pair, 0)

        @pl.when(n_my % 2 == 1)
        def _epilogue_odd():
            launch_adds(idx_a, acc_a, s0a, sma)

            @pl.when(n_my > 1)
            def _finish_b():
                finish(idx_b, acc_b, smb, chunk0 + n_my - 2)

            finish(idx_a, acc_a, sma, chunk0 + n_my - 1)

        @pl.when((n_my % 2 == 0) & (n_my > 0))
        def _epilogue_even():
            finish(idx_b, acc_b, smb, chunk0 + n_my - 1)

    return body(idx_r, features)


def kernel(nodes, neigh_idx, num_sample, features):
    B, S = neigh_idx.shape
    del nodes, num_sample
    C = 448
    per = _NS * (_N0 + _N1) * C
    BP = ((B + per - 1) // per) * per
    idx = neigh_idx.astype(jnp.int32)
    if BP != B:
        idx = jnp.concatenate([idx, jnp.zeros((BP - B, S), jnp.int32)])
    idx_r = jnp.reshape(
        jnp.transpose(jnp.reshape(idx, (BP // C, C, S)), (0, 2, 1)), (-1,)
    )
    return _gather_mean(idx_r, features, B, C, S)


# speedup vs baseline: 1.0080x; 1.0002x over previous
import functools

import jax
import jax.numpy as jnp
from jax import lax
from jax.experimental import pallas as pl
from jax.experimental.pallas import tpu as pltpu
from jax.experimental.pallas import tpu_sc as plsc

_NC = 2
_NS = 16
_NW = _NC * _NS
_LANES = 16
_N0 = 6
_N1 = 1
_K = 0


@functools.partial(jax.jit, static_argnums=(2, 3, 4))
def _gather_mean(idx_r, features, B, chunk_rows, S):
    N, D = features.shape
    C = chunk_rows
    BP = (idx_r.shape[0] // S)
    inv_s = jnp.float32(1.0 / S)
    tail = B % C

    mesh = plsc.VectorSubcoreMesh(
        core_axis_name="c", subcore_axis_name="s",
        num_cores=_NC, num_subcores=_NS,
    )

    @functools.partial(
        pl.kernel,
        mesh=mesh,
        out_type=jax.ShapeDtypeStruct((B, D), jnp.float32),
        scratch_types=[
            pltpu.VMEM((S * C,), jnp.int32),
            pltpu.VMEM((S * C,), jnp.int32),
            pltpu.VMEM((C, D), jnp.float32),
            pltpu.VMEM((C, D), jnp.float32),
            pltpu.SemaphoreType.DMA,
            pltpu.SemaphoreType.DMA,
            pltpu.SemaphoreType.DMA,
            pltpu.SemaphoreType.DMA,
        ],
    )
    def body(idx_hbm, feat_hbm, out_hbm,
             idx_a, idx_b, acc_a, acc_b, s0a, s0b, sma, smb):
        c = lax.axis_index("c")
        s = lax.axis_index("s")
        n_my = jnp.where(
            c == 0,
            _N0 + jnp.where(s < _K, 1, 0),
            jnp.where(s < _NS - _K, _N1, 0),
        )
        chunk0 = jnp.where(
            c == 0,
            s * _N0 + jnp.minimum(s, _K),
            _NS * _N0 + _K + s * _N1,
        )

        def prep(i_ref, a_ref, s0, gid):
            pltpu.sync_copy(idx_hbm.at[pl.ds(gid * (S * C), S * C)], i_ref)
            pltpu.async_copy(feat_hbm.at[i_ref.at[pl.ds(0, C)]], a_ref, s0)

        def launch_adds(i_ref, a_ref, s0, sm):
            pltpu.make_async_copy(
                feat_hbm.at[i_ref.at[pl.ds(0, C)]], a_ref, s0
            ).wait()
            for j in range(1, S):
                pltpu.async_copy(
                    feat_hbm.at[i_ref.at[pl.ds(j * C, C)]], a_ref, sm,
                    add=True,
                )

        def finish(i_ref, a_ref, sm, gid):
            for j in range(1, S):
                pltpu.make_async_copy(
                    feat_hbm.at[i_ref.at[pl.ds(j * C, C)]], a_ref, sm
                ).wait()

            def scale_rows(r2, carry2):
                for dr in range(2):
                    r = 2 * r2 + dr
                    for k in range(D // _LANES):
                        col = pl.ds(k * _LANES, _LANES)
                        a_ref[r, col] = a_ref[r, col] * inv_s
                return carry2

            lax.fori_loop(0, C // 2, scale_rows, 0)
            row0 = gid * C

            @pl.when(row0 + C <= B)
            def _full():
                pltpu.sync_copy(a_ref, out_hbm.at[pl.ds(row0, C)])

            if tail:
                @pl.when(row0 == B - tail)
                def _partial():
                    pltpu.sync_copy(
                        a_ref.at[pl.ds(0, tail)],
                        out_hbm.at[pl.ds(B - tail, tail)],
                    )

        @pl.when(n_my > 0)
        def _prologue():
            prep(idx_a, acc_a, s0a, chunk0)

        def pair(i, carry):
            g0 = 2 * i
            launch_adds(idx_a, acc_a, s0a, sma)

            @pl.when(i > 0)
            def _finish_prev():
                finish(idx_b, acc_b, smb, chunk0 + g0 - 1)

            prep(idx_b, acc_b, s0b, chunk0 + g0 + 1)
            launch_adds(idx_b, acc_b, s0b, smb)
            finish(idx_a, acc_a, sma, chunk0 + g0)

            @pl.when(g0 + 2 < n_my)
            def _prep_next():
                prep(idx_a, acc_a, s0a, chunk0 + g0 + 2)

            return carry

        lax.fori_loop(0, n_my // 2, pair, 0)

        @pl.when(n_my % 2 == 1)
        def _epilogue_odd():
            launch_adds(idx_a, acc_a, s0a, sma)

            @pl.when(n_my > 1)
            def _finish_b():
                finish(idx_b, acc_b, smb, chunk0 + n_my - 2)

            finish(idx_a, acc_a, sma, chunk0 + n_my - 1)

        @pl.when((n_my % 2 == 0) & (n_my > 0))
        def _epilogue_even():
            finish(idx_b, acc_b, smb, chunk0 + n_my - 1)

    return body(idx_r, features)


def kernel(nodes, neigh_idx, num_sample, features):
    B, S = neigh_idx.shape
    del nodes, num_sample
    C = 448
    per = _NS * (_N0 + _N1) * C
    BP = ((B + per - 1) // per) * per
    idx = neigh_idx.astype(jnp.int32)
    if BP != B:
        idx = jnp.concatenate([idx, jnp.zeros((BP - B, S), jnp.int32)])
    idx_r = jnp.reshape(
        jnp.transpose(jnp.reshape(idx, (BP // C, C, S)), (0, 2, 1)), (-1,)
    )
    return _gather_mean(idx_r, features, B, C, S)


# speedup vs baseline: 1.0096x; 1.0015x over previous
import functools

import jax
import jax.numpy as jnp
from jax import lax
from jax.experimental import pallas as pl
from jax.experimental.pallas import tpu as pltpu
from jax.experimental.pallas import tpu_sc as plsc

_NC = 2
_NS = 16
_NW = _NC * _NS
_LANES = 16
_N0 = 6
_N1 = 1
_K = 0


@functools.partial(jax.jit, static_argnums=(2, 3, 4))
def _gather_mean(idx_r, features, B, chunk_rows, S):
    N, D = features.shape
    C = chunk_rows
    BP = (idx_r.shape[0] // S)
    inv_s = jnp.float32(1.0 / S)
    tail = B % C

    mesh = plsc.VectorSubcoreMesh(
        core_axis_name="c", subcore_axis_name="s",
        num_cores=_NC, num_subcores=_NS,
    )

    @functools.partial(
        pl.kernel,
        mesh=mesh,
        out_type=jax.ShapeDtypeStruct((B, D), jnp.float32),
        scratch_types=[
            pltpu.VMEM((S * C,), jnp.int32),
            pltpu.VMEM((S * C,), jnp.int32),
            pltpu.VMEM((C, D), jnp.float32),
            pltpu.VMEM((C, D), jnp.float32),
            pltpu.SemaphoreType.DMA,
            pltpu.SemaphoreType.DMA,
            pltpu.SemaphoreType.DMA,
            pltpu.SemaphoreType.DMA,
        ],
    )
    def body(idx_hbm, feat_hbm, out_hbm,
             idx_a, idx_b, acc_a, acc_b, s0a, s0b, sma, smb):
        c = lax.axis_index("c")
        s = lax.axis_index("s")
        n_my = jnp.where(
            c == 0,
            _N0 + jnp.where(s < _K, 1, 0),
            jnp.where(s < _NS - _K, _N1, 0),
        )
        chunk0 = jnp.where(
            c == 0,
            s * _N0 + jnp.minimum(s, _K),
            _NS * _N0 + _K + s * _N1,
        )

        def prep(i_ref, a_ref, s0, gid):
            pltpu.sync_copy(idx_hbm.at[pl.ds(gid * (S * C), S * C)], i_ref)
            pltpu.async_copy(feat_hbm.at[i_ref.at[pl.ds(0, C)]], a_ref, s0)

        def launch_adds(i_ref, a_ref, s0, sm):
            pltpu.make_async_copy(
                feat_hbm.at[i_ref.at[pl.ds(0, C)]], a_ref, s0
            ).wait()
            for j in range(1, S):
                pltpu.async_copy(
                    feat_hbm.at[i_ref.at[pl.ds(j * C, C)]], a_ref, sm,
                    add=True,
                )

        def finish(i_ref, a_ref, sm, gid):
            for j in range(1, S):
                pltpu.make_async_copy(
                    feat_hbm.at[i_ref.at[pl.ds(j * C, C)]], a_ref, sm
                ).wait()

            def scale_row(r, carry2):
                for k in range(D // _LANES):
                    col = pl.ds(k * _LANES, _LANES)
                    a_ref[r, col] = a_ref[r, col] * inv_s
                return carry2

            lax.fori_loop(0, C, scale_row, 0)
            row0 = gid * C

            @pl.when(row0 + C <= B)
            def _full():
                pltpu.sync_copy(a_ref, out_hbm.at[pl.ds(row0, C)])

            if tail:
                @pl.when(row0 == B - tail)
                def _partial():
                    pltpu.sync_copy(
                        a_ref.at[pl.ds(0, tail)],
                        out_hbm.at[pl.ds(B - tail, tail)],
                    )

        @pl.when(n_my > 0)
        def _prologue():
            prep(idx_a, acc_a, s0a, chunk0)

        def pair(i, carry):
            g0 = 2 * i
            launch_adds(idx_a, acc_a, s0a, sma)

            @pl.when(i > 0)
            def _finish_prev():
                finish(idx_b, acc_b, smb, chunk0 + g0 - 1)

            prep(idx_b, acc_b, s0b, chunk0 + g0 + 1)
            launch_adds(idx_b, acc_b, s0b, smb)
            finish(idx_a, acc_a, sma, chunk0 + g0)

            @pl.when(g0 + 2 < n_my)
            def _prep_next():
                prep(idx_a, acc_a, s0a, chunk0 + g0 + 2)

            return carry

        lax.fori_loop(0, n_my // 2, pair, 0)

        @pl.when(n_my % 2 == 1)
        def _epilogue_odd():
            launch_adds(idx_a, acc_a, s0a, sma)

            @pl.when(n_my > 1)
            def _finish_b():
                finish(idx_b, acc_b, smb, chunk0 + n_my - 2)

            finish(idx_a, acc_a, sma, chunk0 + n_my - 1)

        @pl.when((n_my % 2 == 0) & (n_my > 0))
        def _epilogue_even():
            finish(idx_b, acc_b, smb, chunk0 + n_my - 1)

    return body(idx_r, features)


def kernel(nodes, neigh_idx, num_sample, features):
    B, S = neigh_idx.shape
    del nodes, num_sample
    C = 448
    per = _NS * (_N0 + _N1) * C
    BP = ((B + per - 1) // per) * per
    idx = neigh_idx.astype(jnp.int32)
    if BP != B:
        idx = jnp.concatenate([idx, jnp.zeros((BP - B, S), jnp.int32)])
    idx_r = jnp.reshape(
        jnp.transpose(jnp.reshape(idx, (BP // C, C, S)), (0, 2, 1)), (-1,)
    )
    return _gather_mean(idx_r, features, B, C, S)
